# initial kernel scaffold (unmeasured)
import jax
import jax.numpy as jnp
from jax import lax
from jax.experimental import pallas as pl
from jax.experimental.pallas import tpu as pltpu


def kernel(
    x,
):
    def body(*refs):
        pass

    out_shape = jax.ShapeDtypeStruct(..., jnp.float32)
    return pl.pallas_call(body, out_shape=out_shape)(...)


# baseline (device time: 10581 ns/iter reference)
import jax
import jax.numpy as jnp
from jax import lax
from jax.experimental import pallas as pl
from jax.experimental.pallas import tpu as pltpu


def kernel(x):
    _, m, n = x.shape

    def body(x_ref, out_ref, send_ref, recv_ref, send_sems, recv_sems):
        my = lax.axis_index("i")
        p0 = jnp.bitwise_xor(my, 1)
        p1 = 3 - my

        barrier_sem = pltpu.get_barrier_semaphore()
        for p in (p0, p1):
            pl.semaphore_signal(
                barrier_sem, inc=1,
                device_id=(p,), device_id_type=pl.DeviceIdType.MESH,
            )
        pl.semaphore_wait(barrier_sem, 2)

        send_ref[0] = x_ref[0].astype(jnp.bfloat16)
        rdma0 = pltpu.make_async_remote_copy(
            src_ref=send_ref.at[0],
            dst_ref=recv_ref.at[0],
            send_sem=send_sems.at[0],
            recv_sem=recv_sems.at[0],
            device_id=(p0,),
            device_id_type=pl.DeviceIdType.MESH,
        )
        rdma0.start()
        rdma0.wait()

        send_ref[1] = send_ref[0] + recv_ref[0]
        rdma1 = pltpu.make_async_remote_copy(
            src_ref=send_ref.at[1],
            dst_ref=recv_ref.at[1],
            send_sem=send_sems.at[1],
            recv_sem=recv_sems.at[1],
            device_id=(p1,),
            device_id_type=pl.DeviceIdType.MESH,
        )
        rdma1.start()
        rdma1.wait()

        out_ref[:, :] = (send_ref[1] + recv_ref[1]).astype(jnp.float32)

    return pl.pallas_call(
        body,
        out_shape=jax.ShapeDtypeStruct((m, n), jnp.float32),
        in_specs=[pl.BlockSpec(memory_space=pltpu.VMEM)],
        out_specs=pl.BlockSpec(memory_space=pltpu.VMEM),
        scratch_shapes=[
            pltpu.VMEM((2, m, n), jnp.bfloat16),
            pltpu.VMEM((2, m, n), jnp.bfloat16),
            pltpu.SemaphoreType.DMA((2,)),
            pltpu.SemaphoreType.DMA((2,)),
        ],
        compiler_params=pltpu.CompilerParams(collective_id=0),
    )(x)


# device time: 8958 ns/iter; 1.1812x vs baseline; 1.1812x over previous
import jax
import jax.numpy as jnp
from jax import lax
from jax.experimental import pallas as pl
from jax.experimental.pallas import tpu as pltpu


def kernel(x):
    _, m, n = x.shape
    m2 = m // 2

    def body(x_ref, out_ref, send_a, recv_a, send_b, recv_b, ssems, rsems):
        my = lax.axis_index("i")
        p0 = jnp.bitwise_xor(my, 1)
        p1 = 3 - my

        barrier_sem = pltpu.get_barrier_semaphore()
        for p in (p0, p1):
            pl.semaphore_signal(
                barrier_sem, inc=1,
                device_id=(p,), device_id_type=pl.DeviceIdType.MESH,
            )
        pl.semaphore_wait(barrier_sem, 2)

        xb = x_ref[0].astype(jnp.bfloat16)
        send_a[0] = xb[:m2]
        send_b[0] = xb[m2:]

        rdma_a0 = pltpu.make_async_remote_copy(
            src_ref=send_a.at[0], dst_ref=recv_a.at[0],
            send_sem=ssems.at[0, 0], recv_sem=rsems.at[0, 0],
            device_id=(p0,), device_id_type=pl.DeviceIdType.MESH,
        )
        rdma_b0 = pltpu.make_async_remote_copy(
            src_ref=send_b.at[0], dst_ref=recv_b.at[0],
            send_sem=ssems.at[1, 0], recv_sem=rsems.at[1, 0],
            device_id=(p1,), device_id_type=pl.DeviceIdType.MESH,
        )
        rdma_a0.start()
        rdma_b0.start()
        rdma_a0.wait_recv()
        rdma_b0.wait_recv()

        send_a[1] = send_a[0] + recv_a[0]
        send_b[1] = send_b[0] + recv_b[0]

        rdma_a1 = pltpu.make_async_remote_copy(
            src_ref=send_a.at[1], dst_ref=recv_a.at[1],
            send_sem=ssems.at[0, 1], recv_sem=rsems.at[0, 1],
            device_id=(p1,), device_id_type=pl.DeviceIdType.MESH,
        )
        rdma_b1 = pltpu.make_async_remote_copy(
            src_ref=send_b.at[1], dst_ref=recv_b.at[1],
            send_sem=ssems.at[1, 1], recv_sem=rsems.at[1, 1],
            device_id=(p0,), device_id_type=pl.DeviceIdType.MESH,
        )
        rdma_a1.start()
        rdma_b1.start()
        rdma_a1.wait_recv()
        rdma_b1.wait_recv()

        out_ref[:m2, :] = (send_a[1] + recv_a[1]).astype(jnp.float32)
        out_ref[m2:, :] = (send_b[1] + recv_b[1]).astype(jnp.float32)

        rdma_a0.wait_send()
        rdma_b0.wait_send()
        rdma_a1.wait_send()
        rdma_b1.wait_send()

    return pl.pallas_call(
        body,
        out_shape=jax.ShapeDtypeStruct((m, n), jnp.float32),
        in_specs=[pl.BlockSpec(memory_space=pltpu.VMEM)],
        out_specs=pl.BlockSpec(memory_space=pltpu.VMEM),
        scratch_shapes=[
            pltpu.VMEM((2, m2, n), jnp.bfloat16),
            pltpu.VMEM((2, m2, n), jnp.bfloat16),
            pltpu.VMEM((2, m2, n), jnp.bfloat16),
            pltpu.VMEM((2, m2, n), jnp.bfloat16),
            pltpu.SemaphoreType.DMA((2, 2)),
            pltpu.SemaphoreType.DMA((2, 2)),
        ],
        compiler_params=pltpu.CompilerParams(collective_id=0),
    )(x)
